# baseline (speedup 1.0000x reference)
import functools

import jax
import jax.numpy as jnp
from jax import lax
from jax.experimental import pallas as pl
from jax.experimental.pallas import tpu as pltpu
from jax.experimental.pallas import tpu_sc as plsc

N = 10000
E = 320000
H = 128
L = 4
N_GRAPHS = 64
OUT_DIM_P1 = 65

NC = 2
NS = 16
NW = NC * NS
N_PAD = 10240
E_PAD = 327680
DUMP = N
ROWS_PER_TILE = N_PAD // NW
EDGES_PER_TILE = E_PAD // NW
ECH = 128
AGG_ECH = 128
AGG_NBUF = 2
T0_CHUNKS = 128
T1_CHUNKS = 32
NCH = 64
STRIPE = N_PAD // NS

def _zero_vmem_2d(ref, nrows):
    def body(r, _):
        for j in range(H // 16):
            ref[r, pl.ds(j * 16, 16)] = jnp.zeros((16,), jnp.float32)
        return 0
    lax.fori_loop(0, nrows, body, 0)


def _embed_deg_body(x0_hbm, x1_hbm, dst_hbm, ktab_hbm, vtab_hbm,
               h0_hbm, deg_hbm,
               ik_v, iv_v, krows_v, vrows_v, dst_v, ones_v, zdeg_v, deg_sh, sem):
    c = lax.axis_index("c")
    s = lax.axis_index("s")
    wid = s * NC + c

    def fill(r, _):
        ones_v[pl.ds(r * 16, 16)] = jnp.full((16,), 1.0, jnp.float32)
        return 0
    lax.fori_loop(0, ECH // 16, fill, 0)

    def zfill(r, _):
        zdeg_v[pl.ds(r * 16, 16)] = jnp.zeros((16,), jnp.float32)
        return 0
    lax.fori_loop(0, STRIPE // 16, zfill, 0)

    pltpu.sync_copy(zdeg_v, deg_sh.at[pl.ds(s * STRIPE, STRIPE)])
    plsc.subcore_barrier()

    for t in range(ROWS_PER_TILE // NCH):
        base = wid * ROWS_PER_TILE + t * NCH
        pltpu.sync_copy(x0_hbm.at[pl.ds(base, NCH)], ik_v)
        pltpu.sync_copy(x1_hbm.at[pl.ds(base, NCH)], iv_v)
        pltpu.async_copy(ktab_hbm.at[ik_v], krows_v, sem).wait()
        pltpu.async_copy(vtab_hbm.at[iv_v], vrows_v, sem).wait()

        def addrow(r, _):
            for j in range(H // 16):
                sl = pl.ds(j * 16, 16)
                krows_v[r, sl] = krows_v[r, sl] + vrows_v[r, sl]
            return 0
        lax.fori_loop(0, NCH, addrow, 0)
        pltpu.sync_copy(krows_v, h0_hbm.at[pl.ds(base, NCH)])

    def deg_body(t, _):
        base = pl.multiple_of(wid * EDGES_PER_TILE + t * ECH, 8)
        pltpu.sync_copy(dst_hbm.at[pl.ds(base, ECH)], dst_v)
        pltpu.sync_copy(ones_v, deg_sh.at[dst_v], add=True)
        return 0
    lax.fori_loop(0, EDGES_PER_TILE // ECH, deg_body, 0)

    plsc.subcore_barrier()
    pltpu.sync_copy(deg_sh.at[pl.ds(s * STRIPE, STRIPE)],
                    deg_hbm.at[c, pl.ds(s * STRIPE, STRIPE)])


SUB_CHUNKS = E_PAD // (NS * AGG_ECH)
assert T0_CHUNKS + T1_CHUNKS == SUB_CHUNKS


def _agg_body(h_hbm, srcr_hbm, dstr_hbm, agg_hbm, *scr):
    srcv = scr[0:AGG_NBUF]
    dstv = scr[AGG_NBUF:2 * AGG_NBUF]
    rows = scr[2 * AGG_NBUF:3 * AGG_NBUF]
    agg_sh = scr[3 * AGG_NBUF]
    gsem = scr[3 * AGG_NBUF + 1:3 * AGG_NBUF + 1 + AGG_NBUF]
    ssem = scr[3 * AGG_NBUF + 1 + AGG_NBUF:3 * AGG_NBUF + 1 + 2 * AGG_NBUF]
    c = lax.axis_index("c")
    s = lax.axis_index("s")

    _zero_vmem_2d(rows[0], AGG_ECH)
    for q in range(STRIPE // AGG_ECH):
        pltpu.sync_copy(rows[0], agg_sh.at[pl.ds(s * STRIPE + q * AGG_ECH,
                                                 AGG_ECH)])
    plsc.subcore_barrier()

    def run_ring(nsteps, base):
        for b in range(AGG_NBUF - 1):
            pltpu.sync_copy(srcr_hbm.at[s, base + b], srcv[b])
            pltpu.sync_copy(dstr_hbm.at[s, base + b], dstv[b])
            pltpu.async_copy(h_hbm.at[srcv[b]], rows[b], gsem[b])

        def body(q, _):
            for j in range(AGG_NBUF):
                t = q * AGG_NBUF + j
                pb = (j + AGG_NBUF - 1) % AGG_NBUF

                @pl.when(jnp.logical_and(t >= 1, t + AGG_NBUF - 1 < nsteps))
                def _():
                    pltpu.make_async_copy(rows[pb], agg_sh.at[dstv[pb]],
                                          ssem[pb]).wait()

                @pl.when(t + AGG_NBUF - 1 < nsteps)
                def _():
                    pltpu.sync_copy(srcr_hbm.at[s, base + t + AGG_NBUF - 1],
                                    srcv[pb])
                    pltpu.sync_copy(dstr_hbm.at[s, base + t + AGG_NBUF - 1],
                                    dstv[pb])
                    pltpu.async_copy(h_hbm.at[srcv[pb]], rows[pb], gsem[pb])
                pltpu.make_async_copy(h_hbm.at[srcv[j]], rows[j], gsem[j]).wait()
                pltpu.async_copy(rows[j], agg_sh.at[dstv[j]], ssem[j], add=True)
            return 0
        lax.fori_loop(0, nsteps // AGG_NBUF, body, 0)

        for b in range(AGG_NBUF):
            pltpu.make_async_copy(rows[b], agg_sh.at[dstv[b]], ssem[b]).wait()

    @pl.when(c == 0)
    def _():
        run_ring(T0_CHUNKS, 0)

    @pl.when(c == 1)
    def _():
        run_ring(T1_CHUNKS, T0_CHUNKS)

    plsc.subcore_barrier()
    pltpu.sync_copy(agg_sh.at[pl.ds(s * STRIPE, STRIPE)],
                    agg_hbm.at[c, pl.ds(s * STRIPE, STRIPE)])


@functools.cache
def _sc_kernels():
    mesh = plsc.VectorSubcoreMesh(core_axis_name="c", subcore_axis_name="s")
    embed_deg = functools.partial(
        pl.kernel,
        mesh=mesh,
        out_type=[
            jax.ShapeDtypeStruct((N_PAD, H), jnp.float32),
            jax.ShapeDtypeStruct((NC, N_PAD), jnp.float32),
        ],
        scratch_types=[
            pltpu.VMEM((NCH,), jnp.int32),
            pltpu.VMEM((NCH,), jnp.int32),
            pltpu.VMEM((NCH, H), jnp.float32),
            pltpu.VMEM((NCH, H), jnp.float32),
            pltpu.VMEM((ECH,), jnp.int32),
            pltpu.VMEM((ECH,), jnp.float32),
            pltpu.VMEM((STRIPE,), jnp.float32),
            pltpu.VMEM_SHARED((N_PAD,), jnp.float32),
            pltpu.SemaphoreType.DMA,
        ],
    )(_embed_deg_body)
    agg = functools.partial(
        pl.kernel,
        mesh=mesh,
        out_type=jax.ShapeDtypeStruct((NC, N_PAD, H), jnp.float32),
        scratch_types=(
            [pltpu.VMEM((AGG_ECH,), jnp.int32)] * (2 * AGG_NBUF)
            + [pltpu.VMEM((AGG_ECH, H), jnp.float32)] * AGG_NBUF
            + [pltpu.VMEM_SHARED((N_PAD, H), jnp.float32)]
            + [pltpu.SemaphoreType.DMA] * (2 * AGG_NBUF)
        ),
    )(_agg_body)
    return embed_deg, agg


BR = 512


def _layer_body(h_ref, agg_ref, dinv_ref, ws_ref, wn_ref, g_ref, b_ref, o_ref):
    h = h_ref[...]
    a = (agg_ref[0] + agg_ref[1]) * dinv_ref[...]
    nh = jnp.dot(h, ws_ref[...], preferred_element_type=jnp.float32,
                 precision=lax.Precision.HIGHEST)
    nh = nh + jnp.dot(a, wn_ref[...], preferred_element_type=jnp.float32,
                      precision=lax.Precision.HIGHEST)
    o = h + jnp.maximum(nh, 0.0)
    mu = jnp.mean(o, axis=-1, keepdims=True)
    var = jnp.mean((o - mu) ** 2, axis=-1, keepdims=True)
    o_ref[...] = (o - mu) * lax.rsqrt(var + 1e-5) * g_ref[...] + b_ref[...]


_layer = pl.pallas_call(
    _layer_body,
    grid=(N_PAD // BR,),
    in_specs=[
        pl.BlockSpec((BR, H), lambda i: (i, 0)),
        pl.BlockSpec((NC, BR, H), lambda i: (0, i, 0)),
        pl.BlockSpec((BR, 1), lambda i: (i, 0)),
        pl.BlockSpec((H, H), lambda i: (0, 0)),
        pl.BlockSpec((H, H), lambda i: (0, 0)),
        pl.BlockSpec((1, H), lambda i: (0, 0)),
        pl.BlockSpec((1, H), lambda i: (0, 0)),
    ],
    out_specs=pl.BlockSpec((BR, H), lambda i: (i, 0)),
    out_shape=jax.ShapeDtypeStruct((N_PAD, H), jnp.float32),
)


BD = 2048


def _readout_body(rid_ref, h_ref, wout_ref, o_ref, acc_ref):
    i = pl.program_id(0)

    @pl.when(i == 0)
    def _():
        acc_ref[...] = jnp.zeros_like(acc_ref)

    rows = lax.broadcasted_iota(jnp.int32, (1, BD), 1) + i * BD
    oh = (rid_ref[...] == rows).astype(jnp.float32)
    acc_ref[...] += jnp.dot(oh, h_ref[...], preferred_element_type=jnp.float32,
                            precision=lax.Precision.HIGHEST)

    @pl.when(i == pl.num_programs(0) - 1)
    def _():
        o_ref[...] = jnp.dot(acc_ref[...], wout_ref[...],
                             preferred_element_type=jnp.float32,
                             precision=lax.Precision.HIGHEST)


_readout = pl.pallas_call(
    _readout_body,
    grid=(N_PAD // BD,),
    in_specs=[
        pl.BlockSpec((N_GRAPHS, 1), lambda i: (0, 0)),
        pl.BlockSpec((BD, H), lambda i: (i, 0)),
        pl.BlockSpec((H, OUT_DIM_P1), lambda i: (0, 0)),
    ],
    out_specs=pl.BlockSpec((N_GRAPHS, OUT_DIM_P1), lambda i: (0, 0)),
    out_shape=jax.ShapeDtypeStruct((N_GRAPHS, OUT_DIM_P1), jnp.float32),
    scratch_shapes=[pltpu.VMEM((N_GRAPHS, H), jnp.float32)],
)


def kernel(x, edge_index, batch, root_mask, key_table, val_table,
           W_self, W_nbr, gamma, beta, W_out):
    del batch
    src = edge_index[0].astype(jnp.int32)
    dst = edge_index[1].astype(jnp.int32)
    n_dump = N_PAD - N
    dump_row = DUMP + (jnp.arange(E, dtype=jnp.int32) % n_dump)
    dstp = jnp.where(src == dst, dump_row, dst)
    pad_e = E_PAD - E
    pad_dump = DUMP + (jnp.arange(pad_e, dtype=jnp.int32) % n_dump)
    src_p = jnp.concatenate([src, jnp.zeros((pad_e,), jnp.int32)])
    dst_p = jnp.concatenate([dstp, pad_dump])
    pad_n = N_PAD - N
    x0 = jnp.concatenate([x[:, 0].astype(jnp.int32), jnp.zeros((pad_n,), jnp.int32)])
    x1 = jnp.concatenate([x[:, 1].astype(jnp.int32), jnp.zeros((pad_n,), jnp.int32)])
    root_idx = jnp.nonzero(root_mask, size=N_GRAPHS)[0].astype(jnp.int32)
    root_idx = root_idx.reshape(N_GRAPHS, 1)

    _embed_deg, _agg = _sc_kernels()
    h, degp = _embed_deg(x0, x1, dst_p, key_table, val_table)
    dinv = (1.0 / jnp.clip(degp[0] + degp[1], 1.0, None)).reshape(N_PAD, 1)

    srcr = src_p.reshape(NS, SUB_CHUNKS, AGG_ECH)
    dstr = dst_p.reshape(NS, SUB_CHUNKS, AGG_ECH)
    for i in range(L):
        aggp = _agg(h, srcr, dstr)
        h = _layer(h, aggp, dinv, W_self[i], W_nbr[i],
                   gamma[i].reshape(1, H), beta[i].reshape(1, H))

    return _readout(root_idx, h, W_out)


# speedup vs baseline: 1.0478x; 1.0478x over previous
import functools

import jax
import jax.numpy as jnp
from jax import lax
from jax.experimental import pallas as pl
from jax.experimental.pallas import tpu as pltpu
from jax.experimental.pallas import tpu_sc as plsc

N = 10000
E = 320000
H = 128
L = 4
N_GRAPHS = 64
OUT_DIM_P1 = 65

NC = 2
NS = 16
NW = NC * NS
N_PAD = 10240
E_PAD = 327680
DUMP = N
ROWS_PER_TILE = N_PAD // NW
EDGES_PER_TILE = E_PAD // NW
ECH = 128
AGG_ECH = 128
AGG_NBUF = 2
T0_CHUNKS = 124
T1_CHUNKS = 36
NCH = 64
STRIPE = N_PAD // NS

def _zero_vmem_2d(ref, nrows):
    def body(r, _):
        for j in range(H // 16):
            ref[r, pl.ds(j * 16, 16)] = jnp.zeros((16,), jnp.float32)
        return 0
    lax.fori_loop(0, nrows, body, 0)


def _embed_deg_body(x0_hbm, x1_hbm, dst_hbm, ktab_hbm, vtab_hbm,
               h0_hbm, deg_hbm,
               ik_v, iv_v, krows_v, vrows_v, dst_v, ones_v, zdeg_v, deg_sh, sem):
    c = lax.axis_index("c")
    s = lax.axis_index("s")
    wid = s * NC + c

    def fill(r, _):
        ones_v[pl.ds(r * 16, 16)] = jnp.full((16,), 1.0, jnp.float32)
        return 0
    lax.fori_loop(0, ECH // 16, fill, 0)

    def zfill(r, _):
        zdeg_v[pl.ds(r * 16, 16)] = jnp.zeros((16,), jnp.float32)
        return 0
    lax.fori_loop(0, STRIPE // 16, zfill, 0)

    pltpu.sync_copy(zdeg_v, deg_sh.at[pl.ds(s * STRIPE, STRIPE)])
    plsc.subcore_barrier()

    for t in range(ROWS_PER_TILE // NCH):
        base = wid * ROWS_PER_TILE + t * NCH
        pltpu.sync_copy(x0_hbm.at[pl.ds(base, NCH)], ik_v)
        pltpu.sync_copy(x1_hbm.at[pl.ds(base, NCH)], iv_v)
        pltpu.async_copy(ktab_hbm.at[ik_v], krows_v, sem).wait()
        pltpu.async_copy(vtab_hbm.at[iv_v], vrows_v, sem).wait()

        def addrow(r, _):
            for j in range(H // 16):
                sl = pl.ds(j * 16, 16)
                krows_v[r, sl] = krows_v[r, sl] + vrows_v[r, sl]
            return 0
        lax.fori_loop(0, NCH, addrow, 0)
        pltpu.sync_copy(krows_v, h0_hbm.at[pl.ds(base, NCH)])

    def deg_body(t, _):
        base = pl.multiple_of(wid * EDGES_PER_TILE + t * ECH, 8)
        pltpu.sync_copy(dst_hbm.at[pl.ds(base, ECH)], dst_v)
        pltpu.sync_copy(ones_v, deg_sh.at[dst_v], add=True)
        return 0
    lax.fori_loop(0, EDGES_PER_TILE // ECH, deg_body, 0)

    plsc.subcore_barrier()
    pltpu.sync_copy(deg_sh.at[pl.ds(s * STRIPE, STRIPE)],
                    deg_hbm.at[c, pl.ds(s * STRIPE, STRIPE)])


SUB_CHUNKS = E_PAD // (NS * AGG_ECH)
assert T0_CHUNKS + T1_CHUNKS == SUB_CHUNKS


def _agg_body(h_hbm, srcr_hbm, dstr_hbm, agg_hbm, *scr):
    srcv = scr[0:AGG_NBUF]
    dstv = scr[AGG_NBUF:2 * AGG_NBUF]
    rows = scr[2 * AGG_NBUF:3 * AGG_NBUF]
    agg_sh = scr[3 * AGG_NBUF]
    gsem = scr[3 * AGG_NBUF + 1:3 * AGG_NBUF + 1 + AGG_NBUF]
    ssem = scr[3 * AGG_NBUF + 1 + AGG_NBUF:3 * AGG_NBUF + 1 + 2 * AGG_NBUF]
    c = lax.axis_index("c")
    s = lax.axis_index("s")

    _zero_vmem_2d(rows[0], AGG_ECH)
    for q in range(STRIPE // AGG_ECH):
        pltpu.sync_copy(rows[0], agg_sh.at[pl.ds(s * STRIPE + q * AGG_ECH,
                                                 AGG_ECH)])
    plsc.subcore_barrier()

    def run_ring(nsteps, base):
        for b in range(AGG_NBUF - 1):
            pltpu.sync_copy(srcr_hbm.at[s, base + b], srcv[b])
            pltpu.sync_copy(dstr_hbm.at[s, base + b], dstv[b])
            pltpu.async_copy(h_hbm.at[srcv[b]], rows[b], gsem[b])

        def body(q, _):
            for j in range(AGG_NBUF):
                t = q * AGG_NBUF + j
                pb = (j + AGG_NBUF - 1) % AGG_NBUF

                @pl.when(jnp.logical_and(t >= 1, t + AGG_NBUF - 1 < nsteps))
                def _():
                    pltpu.make_async_copy(rows[pb], agg_sh.at[dstv[pb]],
                                          ssem[pb]).wait()

                @pl.when(t + AGG_NBUF - 1 < nsteps)
                def _():
                    pltpu.sync_copy(srcr_hbm.at[s, base + t + AGG_NBUF - 1],
                                    srcv[pb])
                    pltpu.sync_copy(dstr_hbm.at[s, base + t + AGG_NBUF - 1],
                                    dstv[pb])
                    pltpu.async_copy(h_hbm.at[srcv[pb]], rows[pb], gsem[pb])
                pltpu.make_async_copy(h_hbm.at[srcv[j]], rows[j], gsem[j]).wait()
                pltpu.async_copy(rows[j], agg_sh.at[dstv[j]], ssem[j], add=True)
            return 0
        lax.fori_loop(0, nsteps // AGG_NBUF, body, 0)

        for b in range(AGG_NBUF):
            pltpu.make_async_copy(rows[b], agg_sh.at[dstv[b]], ssem[b]).wait()

    @pl.when(c == 0)
    def _():
        run_ring(T0_CHUNKS, 0)

    @pl.when(c == 1)
    def _():
        run_ring(T1_CHUNKS, T0_CHUNKS)

    plsc.subcore_barrier()
    pltpu.sync_copy(agg_sh.at[pl.ds(s * STRIPE, STRIPE)],
                    agg_hbm.at[c, pl.ds(s * STRIPE, STRIPE)])


@functools.cache
def _sc_kernels():
    mesh = plsc.VectorSubcoreMesh(core_axis_name="c", subcore_axis_name="s")
    embed_deg = functools.partial(
        pl.kernel,
        mesh=mesh,
        out_type=[
            jax.ShapeDtypeStruct((N_PAD, H), jnp.float32),
            jax.ShapeDtypeStruct((NC, N_PAD), jnp.float32),
        ],
        scratch_types=[
            pltpu.VMEM((NCH,), jnp.int32),
            pltpu.VMEM((NCH,), jnp.int32),
            pltpu.VMEM((NCH, H), jnp.float32),
            pltpu.VMEM((NCH, H), jnp.float32),
            pltpu.VMEM((ECH,), jnp.int32),
            pltpu.VMEM((ECH,), jnp.float32),
            pltpu.VMEM((STRIPE,), jnp.float32),
            pltpu.VMEM_SHARED((N_PAD,), jnp.float32),
            pltpu.SemaphoreType.DMA,
        ],
    )(_embed_deg_body)
    agg = functools.partial(
        pl.kernel,
        mesh=mesh,
        out_type=jax.ShapeDtypeStruct((NC, N_PAD, H), jnp.float32),
        scratch_types=(
            [pltpu.VMEM((AGG_ECH,), jnp.int32)] * (2 * AGG_NBUF)
            + [pltpu.VMEM((AGG_ECH, H), jnp.float32)] * AGG_NBUF
            + [pltpu.VMEM_SHARED((N_PAD, H), jnp.float32)]
            + [pltpu.SemaphoreType.DMA] * (2 * AGG_NBUF)
        ),
    )(_agg_body)
    return embed_deg, agg


BR = 512


def _layer_body(h_ref, agg_ref, dinv_ref, ws_ref, wn_ref, g_ref, b_ref, o_ref):
    h = h_ref[...]
    a = (agg_ref[0] + agg_ref[1]) * dinv_ref[...]
    nh = jnp.dot(h, ws_ref[...], preferred_element_type=jnp.float32,
                 precision=lax.Precision.HIGHEST)
    nh = nh + jnp.dot(a, wn_ref[...], preferred_element_type=jnp.float32,
                      precision=lax.Precision.HIGHEST)
    o = h + jnp.maximum(nh, 0.0)
    mu = jnp.mean(o, axis=-1, keepdims=True)
    var = jnp.mean((o - mu) ** 2, axis=-1, keepdims=True)
    o_ref[...] = (o - mu) * lax.rsqrt(var + 1e-5) * g_ref[...] + b_ref[...]


_layer = pl.pallas_call(
    _layer_body,
    grid=(N_PAD // BR,),
    in_specs=[
        pl.BlockSpec((BR, H), lambda i: (i, 0)),
        pl.BlockSpec((NC, BR, H), lambda i: (0, i, 0)),
        pl.BlockSpec((BR, 1), lambda i: (i, 0)),
        pl.BlockSpec((H, H), lambda i: (0, 0)),
        pl.BlockSpec((H, H), lambda i: (0, 0)),
        pl.BlockSpec((1, H), lambda i: (0, 0)),
        pl.BlockSpec((1, H), lambda i: (0, 0)),
    ],
    out_specs=pl.BlockSpec((BR, H), lambda i: (i, 0)),
    out_shape=jax.ShapeDtypeStruct((N_PAD, H), jnp.float32),
)


BD = 2048


def _readout_body(rid_ref, h_ref, wout_ref, o_ref, acc_ref):
    i = pl.program_id(0)

    @pl.when(i == 0)
    def _():
        acc_ref[...] = jnp.zeros_like(acc_ref)

    rows = lax.broadcasted_iota(jnp.int32, (1, BD), 1) + i * BD
    oh = (rid_ref[...] == rows).astype(jnp.float32)
    acc_ref[...] += jnp.dot(oh, h_ref[...], preferred_element_type=jnp.float32,
                            precision=lax.Precision.HIGHEST)

    @pl.when(i == pl.num_programs(0) - 1)
    def _():
        o_ref[...] = jnp.dot(acc_ref[...], wout_ref[...],
                             preferred_element_type=jnp.float32,
                             precision=lax.Precision.HIGHEST)


_readout = pl.pallas_call(
    _readout_body,
    grid=(N_PAD // BD,),
    in_specs=[
        pl.BlockSpec((N_GRAPHS, 1), lambda i: (0, 0)),
        pl.BlockSpec((BD, H), lambda i: (i, 0)),
        pl.BlockSpec((H, OUT_DIM_P1), lambda i: (0, 0)),
    ],
    out_specs=pl.BlockSpec((N_GRAPHS, OUT_DIM_P1), lambda i: (0, 0)),
    out_shape=jax.ShapeDtypeStruct((N_GRAPHS, OUT_DIM_P1), jnp.float32),
    scratch_shapes=[pltpu.VMEM((N_GRAPHS, H), jnp.float32)],
)


def kernel(x, edge_index, batch, root_mask, key_table, val_table,
           W_self, W_nbr, gamma, beta, W_out):
    del batch
    src = edge_index[0].astype(jnp.int32)
    dst = edge_index[1].astype(jnp.int32)
    n_dump = N_PAD - N
    dump_row = DUMP + (jnp.arange(E, dtype=jnp.int32) % n_dump)
    dstp = jnp.where(src == dst, dump_row, dst)
    pad_e = E_PAD - E
    pad_dump = DUMP + (jnp.arange(pad_e, dtype=jnp.int32) % n_dump)
    src_p = jnp.concatenate([src, jnp.zeros((pad_e,), jnp.int32)])
    dst_p = jnp.concatenate([dstp, pad_dump])
    pad_n = N_PAD - N
    x0 = jnp.concatenate([x[:, 0].astype(jnp.int32), jnp.zeros((pad_n,), jnp.int32)])
    x1 = jnp.concatenate([x[:, 1].astype(jnp.int32), jnp.zeros((pad_n,), jnp.int32)])
    root_idx = jnp.nonzero(root_mask, size=N_GRAPHS)[0].astype(jnp.int32)
    root_idx = root_idx.reshape(N_GRAPHS, 1)

    _embed_deg, _agg = _sc_kernels()
    h, degp = _embed_deg(x0, x1, dst_p, key_table, val_table)
    dinv = (1.0 / jnp.clip(degp[0] + degp[1], 1.0, None)).reshape(N_PAD, 1)

    srcr = src_p.reshape(NS, SUB_CHUNKS, AGG_ECH)
    dstr = dst_p.reshape(NS, SUB_CHUNKS, AGG_ECH)
    for i in range(L):
        aggp = _agg(h, srcr, dstr)
        h = _layer(h, aggp, dinv, W_self[i], W_nbr[i],
                   gamma[i].reshape(1, H), beta[i].reshape(1, H))

    return _readout(root_idx, h, W_out)


# speedup vs baseline: 1.0759x; 1.0269x over previous
import functools

import jax
import jax.numpy as jnp
from jax import lax
from jax.experimental import pallas as pl
from jax.experimental.pallas import tpu as pltpu
from jax.experimental.pallas import tpu_sc as plsc

N = 10000
E = 320000
H = 128
L = 4
N_GRAPHS = 64
OUT_DIM_P1 = 65

NC = 2
NS = 16
NW = NC * NS
N_PAD = 10240
E_PAD = 327680
DUMP = N
ROWS_PER_TILE = N_PAD // NW
EDGES_PER_TILE = E_PAD // NW
ECH = 128
AGG_ECH = 128
AGG_NBUF = 2
T0_CHUNKS = 122
T1_CHUNKS = 38
NCH = 64
STRIPE = N_PAD // NS

def _zero_vmem_2d(ref, nrows):
    def body(r, _):
        for j in range(H // 16):
            ref[r, pl.ds(j * 16, 16)] = jnp.zeros((16,), jnp.float32)
        return 0
    lax.fori_loop(0, nrows, body, 0)


def _embed_deg_body(x0_hbm, x1_hbm, dst_hbm, ktab_hbm, vtab_hbm,
               h0_hbm, deg_hbm,
               ik_v, iv_v, krows_v, vrows_v, dst_v, ones_v, zdeg_v, deg_sh, sem):
    c = lax.axis_index("c")
    s = lax.axis_index("s")
    wid = s * NC + c

    def fill(r, _):
        ones_v[pl.ds(r * 16, 16)] = jnp.full((16,), 1.0, jnp.float32)
        return 0
    lax.fori_loop(0, ECH // 16, fill, 0)

    def zfill(r, _):
        zdeg_v[pl.ds(r * 16, 16)] = jnp.zeros((16,), jnp.float32)
        return 0
    lax.fori_loop(0, STRIPE // 16, zfill, 0)

    pltpu.sync_copy(zdeg_v, deg_sh.at[pl.ds(s * STRIPE, STRIPE)])
    plsc.subcore_barrier()

    for t in range(ROWS_PER_TILE // NCH):
        base = wid * ROWS_PER_TILE + t * NCH
        pltpu.sync_copy(x0_hbm.at[pl.ds(base, NCH)], ik_v)
        pltpu.sync_copy(x1_hbm.at[pl.ds(base, NCH)], iv_v)
        pltpu.async_copy(ktab_hbm.at[ik_v], krows_v, sem).wait()
        pltpu.async_copy(vtab_hbm.at[iv_v], vrows_v, sem).wait()

        def addrow(r, _):
            for j in range(H // 16):
                sl = pl.ds(j * 16, 16)
                krows_v[r, sl] = krows_v[r, sl] + vrows_v[r, sl]
            return 0
        lax.fori_loop(0, NCH, addrow, 0)
        pltpu.sync_copy(krows_v, h0_hbm.at[pl.ds(base, NCH)])

    def deg_body(t, _):
        base = pl.multiple_of(wid * EDGES_PER_TILE + t * ECH, 8)
        pltpu.sync_copy(dst_hbm.at[pl.ds(base, ECH)], dst_v)
        pltpu.sync_copy(ones_v, deg_sh.at[dst_v], add=True)
        return 0
    lax.fori_loop(0, EDGES_PER_TILE // ECH, deg_body, 0)

    plsc.subcore_barrier()
    pltpu.sync_copy(deg_sh.at[pl.ds(s * STRIPE, STRIPE)],
                    deg_hbm.at[c, pl.ds(s * STRIPE, STRIPE)])


SUB_CHUNKS = E_PAD // (NS * AGG_ECH)
assert T0_CHUNKS + T1_CHUNKS == SUB_CHUNKS


def _agg_body(h_hbm, srcr_hbm, dstr_hbm, agg_hbm, *scr):
    srcv = scr[0:AGG_NBUF]
    dstv = scr[AGG_NBUF:2 * AGG_NBUF]
    rows = scr[2 * AGG_NBUF:3 * AGG_NBUF]
    agg_sh = scr[3 * AGG_NBUF]
    gsem = scr[3 * AGG_NBUF + 1:3 * AGG_NBUF + 1 + AGG_NBUF]
    ssem = scr[3 * AGG_NBUF + 1 + AGG_NBUF:3 * AGG_NBUF + 1 + 2 * AGG_NBUF]
    c = lax.axis_index("c")
    s = lax.axis_index("s")

    _zero_vmem_2d(rows[0], AGG_ECH)
    for q in range(STRIPE // AGG_ECH):
        pltpu.sync_copy(rows[0], agg_sh.at[pl.ds(s * STRIPE + q * AGG_ECH,
                                                 AGG_ECH)])
    plsc.subcore_barrier()

    def run_ring(nsteps, base):
        for b in range(AGG_NBUF - 1):
            pltpu.sync_copy(srcr_hbm.at[s, base + b], srcv[b])
            pltpu.sync_copy(dstr_hbm.at[s, base + b], dstv[b])
            pltpu.async_copy(h_hbm.at[srcv[b]], rows[b], gsem[b])

        def body(q, _):
            for j in range(AGG_NBUF):
                t = q * AGG_NBUF + j
                pb = (j + AGG_NBUF - 1) % AGG_NBUF

                @pl.when(jnp.logical_and(t >= 1, t + AGG_NBUF - 1 < nsteps))
                def _():
                    pltpu.make_async_copy(rows[pb], agg_sh.at[dstv[pb]],
                                          ssem[pb]).wait()

                @pl.when(t + AGG_NBUF - 1 < nsteps)
                def _():
                    pltpu.sync_copy(srcr_hbm.at[s, base + t + AGG_NBUF - 1],
                                    srcv[pb])
                    pltpu.sync_copy(dstr_hbm.at[s, base + t + AGG_NBUF - 1],
                                    dstv[pb])
                    pltpu.async_copy(h_hbm.at[srcv[pb]], rows[pb], gsem[pb])
                pltpu.make_async_copy(h_hbm.at[srcv[j]], rows[j], gsem[j]).wait()
                pltpu.async_copy(rows[j], agg_sh.at[dstv[j]], ssem[j], add=True)
            return 0
        lax.fori_loop(0, nsteps // AGG_NBUF, body, 0)

        for b in range(AGG_NBUF):
            pltpu.make_async_copy(rows[b], agg_sh.at[dstv[b]], ssem[b]).wait()

    @pl.when(c == 0)
    def _():
        run_ring(T0_CHUNKS, 0)

    @pl.when(c == 1)
    def _():
        run_ring(T1_CHUNKS, T0_CHUNKS)

    plsc.subcore_barrier()
    pltpu.sync_copy(agg_sh.at[pl.ds(s * STRIPE, STRIPE)],
                    agg_hbm.at[c, pl.ds(s * STRIPE, STRIPE)])


@functools.cache
def _sc_kernels():
    mesh = plsc.VectorSubcoreMesh(core_axis_name="c", subcore_axis_name="s")
    embed_deg = functools.partial(
        pl.kernel,
        mesh=mesh,
        out_type=[
            jax.ShapeDtypeStruct((N_PAD, H), jnp.float32),
            jax.ShapeDtypeStruct((NC, N_PAD), jnp.float32),
        ],
        scratch_types=[
            pltpu.VMEM((NCH,), jnp.int32),
            pltpu.VMEM((NCH,), jnp.int32),
            pltpu.VMEM((NCH, H), jnp.float32),
            pltpu.VMEM((NCH, H), jnp.float32),
            pltpu.VMEM((ECH,), jnp.int32),
            pltpu.VMEM((ECH,), jnp.float32),
            pltpu.VMEM((STRIPE,), jnp.float32),
            pltpu.VMEM_SHARED((N_PAD,), jnp.float32),
            pltpu.SemaphoreType.DMA,
        ],
    )(_embed_deg_body)
    agg = functools.partial(
        pl.kernel,
        mesh=mesh,
        out_type=jax.ShapeDtypeStruct((NC, N_PAD, H), jnp.float32),
        scratch_types=(
            [pltpu.VMEM((AGG_ECH,), jnp.int32)] * (2 * AGG_NBUF)
            + [pltpu.VMEM((AGG_ECH, H), jnp.float32)] * AGG_NBUF
            + [pltpu.VMEM_SHARED((N_PAD, H), jnp.float32)]
            + [pltpu.SemaphoreType.DMA] * (2 * AGG_NBUF)
        ),
    )(_agg_body)
    return embed_deg, agg


BR = 512


def _layer_body(h_ref, agg_ref, dinv_ref, ws_ref, wn_ref, g_ref, b_ref, o_ref):
    h = h_ref[...]
    a = (agg_ref[0] + agg_ref[1]) * dinv_ref[...]
    nh = jnp.dot(h, ws_ref[...], preferred_element_type=jnp.float32,
                 precision=lax.Precision.HIGHEST)
    nh = nh + jnp.dot(a, wn_ref[...], preferred_element_type=jnp.float32,
                      precision=lax.Precision.HIGHEST)
    o = h + jnp.maximum(nh, 0.0)
    mu = jnp.mean(o, axis=-1, keepdims=True)
    var = jnp.mean((o - mu) ** 2, axis=-1, keepdims=True)
    o_ref[...] = (o - mu) * lax.rsqrt(var + 1e-5) * g_ref[...] + b_ref[...]


_layer = pl.pallas_call(
    _layer_body,
    grid=(N_PAD // BR,),
    in_specs=[
        pl.BlockSpec((BR, H), lambda i: (i, 0)),
        pl.BlockSpec((NC, BR, H), lambda i: (0, i, 0)),
        pl.BlockSpec((BR, 1), lambda i: (i, 0)),
        pl.BlockSpec((H, H), lambda i: (0, 0)),
        pl.BlockSpec((H, H), lambda i: (0, 0)),
        pl.BlockSpec((1, H), lambda i: (0, 0)),
        pl.BlockSpec((1, H), lambda i: (0, 0)),
    ],
    out_specs=pl.BlockSpec((BR, H), lambda i: (i, 0)),
    out_shape=jax.ShapeDtypeStruct((N_PAD, H), jnp.float32),
)


BD = 2048


def _readout_body(rid_ref, h_ref, wout_ref, o_ref, acc_ref):
    i = pl.program_id(0)

    @pl.when(i == 0)
    def _():
        acc_ref[...] = jnp.zeros_like(acc_ref)

    rows = lax.broadcasted_iota(jnp.int32, (1, BD), 1) + i * BD
    oh = (rid_ref[...] == rows).astype(jnp.float32)
    acc_ref[...] += jnp.dot(oh, h_ref[...], preferred_element_type=jnp.float32,
                            precision=lax.Precision.HIGHEST)

    @pl.when(i == pl.num_programs(0) - 1)
    def _():
        o_ref[...] = jnp.dot(acc_ref[...], wout_ref[...],
                             preferred_element_type=jnp.float32,
                             precision=lax.Precision.HIGHEST)


_readout = pl.pallas_call(
    _readout_body,
    grid=(N_PAD // BD,),
    in_specs=[
        pl.BlockSpec((N_GRAPHS, 1), lambda i: (0, 0)),
        pl.BlockSpec((BD, H), lambda i: (i, 0)),
        pl.BlockSpec((H, OUT_DIM_P1), lambda i: (0, 0)),
    ],
    out_specs=pl.BlockSpec((N_GRAPHS, OUT_DIM_P1), lambda i: (0, 0)),
    out_shape=jax.ShapeDtypeStruct((N_GRAPHS, OUT_DIM_P1), jnp.float32),
    scratch_shapes=[pltpu.VMEM((N_GRAPHS, H), jnp.float32)],
)


def kernel(x, edge_index, batch, root_mask, key_table, val_table,
           W_self, W_nbr, gamma, beta, W_out):
    del batch
    src = edge_index[0].astype(jnp.int32)
    dst = edge_index[1].astype(jnp.int32)
    n_dump = N_PAD - N
    dump_row = DUMP + (jnp.arange(E, dtype=jnp.int32) % n_dump)
    dstp = jnp.where(src == dst, dump_row, dst)
    pad_e = E_PAD - E
    pad_dump = DUMP + (jnp.arange(pad_e, dtype=jnp.int32) % n_dump)
    src_p = jnp.concatenate([src, jnp.zeros((pad_e,), jnp.int32)])
    dst_p = jnp.concatenate([dstp, pad_dump])
    pad_n = N_PAD - N
    x0 = jnp.concatenate([x[:, 0].astype(jnp.int32), jnp.zeros((pad_n,), jnp.int32)])
    x1 = jnp.concatenate([x[:, 1].astype(jnp.int32), jnp.zeros((pad_n,), jnp.int32)])
    root_idx = jnp.nonzero(root_mask, size=N_GRAPHS)[0].astype(jnp.int32)
    root_idx = root_idx.reshape(N_GRAPHS, 1)

    _embed_deg, _agg = _sc_kernels()
    h, degp = _embed_deg(x0, x1, dst_p, key_table, val_table)
    dinv = (1.0 / jnp.clip(degp[0] + degp[1], 1.0, None)).reshape(N_PAD, 1)

    srcr = src_p.reshape(NS, SUB_CHUNKS, AGG_ECH)
    dstr = dst_p.reshape(NS, SUB_CHUNKS, AGG_ECH)
    for i in range(L):
        aggp = _agg(h, srcr, dstr)
        h = _layer(h, aggp, dinv, W_self[i], W_nbr[i],
                   gamma[i].reshape(1, H), beta[i].reshape(1, H))

    return _readout(root_idx, h, W_out)
